# baseline (speedup 1.0000x reference)
import functools

import jax
import jax.numpy as jnp
from jax import lax
from jax.experimental import pallas as pl
from jax.experimental.pallas import tpu as pltpu
from jax.experimental.pallas import tpu_sc as plsc

B = 16384
P = 20
N = 50
DIM = 64
CROW = P + N
SCW = 80
E = 8
NW = 32
EPW = B // NW
NCHUNK = EPW // E
ROWS = E * CROW
PROWS = E * P
NROWS = E * N
NG = 5


def _sc_scores(words, pos1d, neg1d, w_embedding, c_embedding):
    mesh = plsc.VectorSubcoreMesh(core_axis_name="c", subcore_axis_name="s")

    @functools.partial(
        pl.kernel,
        out_type=jax.ShapeDtypeStruct((B, SCW), jnp.float32),
        mesh=mesh,
        compiler_params=pltpu.CompilerParams(needs_layout_passes=False,
                                             use_tc_tiling_on_sc=False),
        scratch_types=[
            pltpu.VMEM((EPW * P,), jnp.int32),
            pltpu.VMEM((EPW * N,), jnp.int32),
            pltpu.VMEM((EPW,), jnp.int32),
            pltpu.VMEM((ROWS, DIM), jnp.float32),
            pltpu.VMEM((ROWS, DIM), jnp.float32),
            pltpu.VMEM((E, DIM), jnp.float32),
            pltpu.VMEM((E, DIM), jnp.float32),
            pltpu.VMEM((E, SCW), jnp.float32),
            pltpu.VMEM((E, SCW), jnp.float32),
        ] + [pltpu.SemaphoreType.DMA] * 14,
    )
    def sc_kernel(words_hbm, pos_hbm, neg_hbm, wtab_hbm, ctab_hbm, out_hbm,
                  pidx_v, nidx_v, widx_v, rows0, rows1, wrows0, wrows1,
                  scores0, scores1, *sems):
        rows_v = (rows0, rows1)
        wrows_v = (wrows0, wrows1)
        scores_v = (scores0, scores1)
        semf = (sems[0:6], sems[6:12])
        semo = (sems[12], sems[13])
        wid = lax.axis_index("s") * 2 + lax.axis_index("c")
        base_e0 = wid * EPW

        def start_fetch(c, slot):
            pltpu.async_copy(
                ctab_hbm.at[pidx_v.at[pl.ds(c * PROWS, PROWS)]],
                rows_v[slot].at[pl.ds(0, PROWS)],
                semf[slot][0],
            )
            pltpu.async_copy(
                ctab_hbm.at[nidx_v.at[pl.ds(c * NROWS, NROWS)]],
                rows_v[slot].at[pl.ds(PROWS, NROWS)],
                semf[slot][1],
            )
            pltpu.async_copy(wtab_hbm.at[widx_v.at[pl.ds(c * E, E)]],
                             wrows_v[slot], semf[slot][2])

        def wait_fetch(slot):
            pltpu.make_async_copy(
                ctab_hbm.at[pl.ds(0, PROWS)],
                rows_v[slot].at[pl.ds(0, PROWS)], semf[slot][0]
            ).wait()
            pltpu.make_async_copy(
                ctab_hbm.at[pl.ds(0, NROWS)],
                rows_v[slot].at[pl.ds(PROWS, NROWS)], semf[slot][1]
            ).wait()
            pltpu.make_async_copy(
                wtab_hbm.at[pl.ds(0, E)], wrows_v[slot], semf[slot][2]
            ).wait()

        def drain_out(slot):
            pltpu.make_async_copy(
                scores_v[slot], out_hbm.at[pl.ds(0, E)], semo[slot]
            ).wait()

        def compute(slot):
            lanes = lax.iota(jnp.int32, 16)

            def elem_body(e, _):
                accs = [jnp.zeros((16,), jnp.float32) for _ in range(NG)]
                pb = e * P
                nb = PROWS + e * N
                rowidx = []
                for g in range(NG):
                    j = jnp.minimum(g * 16 + lanes, CROW - 1)
                    rowidx.append(jnp.where(j < P, pb + j, nb + j - P))
                for k in range(DIM // 16):
                    wchunk = wrows_v[slot][e, pl.ds(k * 16, 16)]
                    for i in range(16):
                        d = k * 16 + i
                        wd = wchunk[i]
                        col_idx = jnp.full((16,), d, jnp.int32)
                        for g in range(NG):
                            col = plsc.load_gather(
                                rows_v[slot], [rowidx[g], col_idx])
                            accs[g] = accs[g] + col * wd
                for g in range(NG):
                    scores_v[slot][e, pl.ds(g * 16, 16)] = accs[g]
                return 0

            lax.fori_loop(0, E, elem_body, 0, unroll=2)

        pltpu.sync_copy(pos_hbm.at[pl.ds(base_e0 * P, EPW * P)], pidx_v)
        pltpu.sync_copy(neg_hbm.at[pl.ds(base_e0 * N, EPW * N)], nidx_v)
        pltpu.sync_copy(words_hbm.at[pl.ds(base_e0, EPW)], widx_v)
        start_fetch(0, 0)
        start_fetch(1, 1)

        def chunk_body(g, _):
            for b in range(2):
                c = g * 2 + b
                wait_fetch(b)

                @pl.when(c >= 2)
                def _():
                    drain_out(b)

                compute(b)
                pltpu.async_copy(
                    scores_v[b],
                    out_hbm.at[pl.ds(base_e0 + c * E, E)],
                    semo[b],
                )

                @pl.when(c + 2 < NCHUNK)
                def _():
                    start_fetch(c + 2, b)
            return 0

        lax.fori_loop(0, NCHUNK // 2, chunk_body, 0, unroll=False)
        drain_out(0)
        drain_out(1)

    return sc_kernel(words, pos1d, neg1d, w_embedding, c_embedding)


def _tc_loss(scores):
    blk = 2048

    def tc_body(s_ref, o_ref):
        s = s_ref[...]
        j = lax.broadcasted_iota(jnp.int32, s.shape, 1)
        pos = jnp.where(j < P, jax.nn.log_sigmoid(s), 0.0).sum(axis=1) / P
        neg = jnp.where((j >= P) & (j < P + N),
                        jax.nn.log_sigmoid(-s), 0.0).sum(axis=1) / N
        o_ref[...] = -(pos + neg)

    return pl.pallas_call(
        tc_body,
        grid=(B // blk,),
        in_specs=[pl.BlockSpec((blk, SCW), lambda i: (i, 0))],
        out_specs=pl.BlockSpec((blk,), lambda i: (i,)),
        out_shape=jax.ShapeDtypeStruct((B,), jnp.float32),
    )(scores)


def kernel(words, pos_contexts, neg_contexts, w_embedding, c_embedding):
    scores = _sc_scores(words, pos_contexts.reshape(-1),
                        neg_contexts.reshape(-1), w_embedding, c_embedding)
    return _tc_loss(scores)


# speedup vs baseline: 1.0086x; 1.0086x over previous
import functools

import jax
import jax.numpy as jnp
from jax import lax
from jax.experimental import pallas as pl
from jax.experimental.pallas import tpu as pltpu
from jax.experimental.pallas import tpu_sc as plsc

B = 16384
P = 20
N = 50
DIM = 64
CROW = P + N
SCW = 80
E = 8
NW = 32
EPW = B // NW
NCHUNK = EPW // E
ROWS = E * CROW
IDXW = 112
IDXROWS = ROWS // IDXW
NG = 5


def _sc_scores(words, ctx, w_embedding, c_embedding):
    mesh = plsc.VectorSubcoreMesh(core_axis_name="c", subcore_axis_name="s")

    @functools.partial(
        pl.kernel,
        out_type=jax.ShapeDtypeStruct((B, SCW), jnp.float32),
        mesh=mesh,
        compiler_params=pltpu.CompilerParams(needs_layout_passes=False,
                                             use_tc_tiling_on_sc=False),
        scratch_types=[
            pltpu.VMEM((EPW * CROW,), jnp.int32),
            pltpu.VMEM((EPW,), jnp.int32),
            pltpu.VMEM((ROWS, DIM), jnp.float32),
            pltpu.VMEM((ROWS, DIM), jnp.float32),
            pltpu.VMEM((E, DIM), jnp.float32),
            pltpu.VMEM((E, DIM), jnp.float32),
            pltpu.VMEM((E, SCW), jnp.float32),
            pltpu.VMEM((E, SCW), jnp.float32),
        ] + [pltpu.SemaphoreType.DMA] * 14,
    )
    def sc_kernel(words_hbm, ctx_hbm, wtab_hbm, ctab_hbm, out_hbm,
                  idx_v, widx_v, rows0, rows1, wrows0, wrows1,
                  scores0, scores1, *sems):
        rows_v = (rows0, rows1)
        wrows_v = (wrows0, wrows1)
        scores_v = (scores0, scores1)
        semf = (sems[0:6], sems[6:12])
        semo = (sems[12], sems[13])
        wid = lax.axis_index("s") * 2 + lax.axis_index("c")
        base_e0 = wid * EPW

        def start_fetch(c, slot):
            off = c * ROWS
            pltpu.async_copy(
                ctab_hbm.at[idx_v.at[pl.ds(off, ROWS)]],
                rows_v[slot],
                semf[slot][0],
            )
            pltpu.async_copy(wtab_hbm.at[widx_v.at[pl.ds(c * E, E)]],
                             wrows_v[slot], semf[slot][IDXROWS])

        def wait_fetch(slot):
            pltpu.make_async_copy(
                ctab_hbm.at[pl.ds(0, ROWS)], rows_v[slot], semf[slot][0]
            ).wait()
            pltpu.make_async_copy(
                wtab_hbm.at[pl.ds(0, E)], wrows_v[slot], semf[slot][IDXROWS]
            ).wait()

        def drain_out(slot):
            pltpu.make_async_copy(
                scores_v[slot], out_hbm.at[pl.ds(0, E)], semo[slot]
            ).wait()

        def compute(slot):
            lanes = lax.iota(jnp.int32, 16)

            def elem_body(e, _):
                accs = [jnp.zeros((16,), jnp.float32) for _ in range(NG)]
                row0 = e * CROW + lanes
                rowidx = [row0 + g * 16 for g in range(NG - 1)]
                rowidx.append(jnp.minimum(row0 + 64, ROWS - 1))
                for k in range(DIM // 16):
                    wchunk = wrows_v[slot][e, pl.ds(k * 16, 16)]
                    for i in range(16):
                        d = k * 16 + i
                        wd = wchunk[i]
                        col_idx = jnp.full((16,), d, jnp.int32)
                        for g in range(NG):
                            col = plsc.load_gather(
                                rows_v[slot], [rowidx[g], col_idx])
                            accs[g] = accs[g] + col * wd
                for g in range(NG):
                    scores_v[slot][e, pl.ds(g * 16, 16)] = accs[g]
                return 0

            lax.fori_loop(0, E, elem_body, 0, unroll=2)

        pltpu.sync_copy(ctx_hbm.at[pl.ds(base_e0 * CROW, EPW * CROW)], idx_v)
        pltpu.sync_copy(words_hbm.at[pl.ds(base_e0, EPW)], widx_v)
        start_fetch(0, 0)
        start_fetch(1, 1)

        def chunk_body(g, _):
            for b in range(2):
                c = g * 2 + b
                wait_fetch(b)

                @pl.when(c >= 2)
                def _():
                    drain_out(b)

                compute(b)
                pltpu.async_copy(
                    scores_v[b],
                    out_hbm.at[pl.ds(base_e0 + c * E, E)],
                    semo[b],
                )

                @pl.when(c + 2 < NCHUNK)
                def _():
                    start_fetch(c + 2, b)
            return 0

        lax.fori_loop(0, NCHUNK // 2, chunk_body, 0, unroll=False)
        drain_out(0)
        drain_out(1)

    return sc_kernel(words, ctx, w_embedding, c_embedding)


def _tc_loss(scores):
    blk = 2048

    def tc_body(s_ref, o_ref):
        s = s_ref[...]
        j = lax.broadcasted_iota(jnp.int32, s.shape, 1)
        pos = jnp.where(j < P, jax.nn.log_sigmoid(s), 0.0).sum(axis=1) / P
        neg = jnp.where((j >= P) & (j < P + N),
                        jax.nn.log_sigmoid(-s), 0.0).sum(axis=1) / N
        o_ref[...] = -(pos + neg)

    return pl.pallas_call(
        tc_body,
        grid=(B // blk,),
        in_specs=[pl.BlockSpec((blk, SCW), lambda i: (i, 0))],
        out_specs=pl.BlockSpec((blk,), lambda i: (i,)),
        out_shape=jax.ShapeDtypeStruct((B,), jnp.float32),
    )(scores)


def kernel(words, pos_contexts, neg_contexts, w_embedding, c_embedding):
    ctx = jnp.concatenate([pos_contexts, neg_contexts], axis=1).reshape(-1)
    scores = _sc_scores(words, ctx, w_embedding, c_embedding)
    return _tc_loss(scores)
